# Initial kernel scaffold; baseline (speedup 1.0000x reference)
import jax
import jax.numpy as jnp
from jax.experimental import pallas as pl


def kernel(edge_index, edge_values, embeds):
    raise NotImplementedError("write your pallas kernel here")


# speedup vs baseline: 4.4154x; 4.4154x over previous
import functools

import jax
import jax.numpy as jnp
from jax import lax
from jax.experimental import pallas as pl
from jax.experimental.pallas import tpu as pltpu
from jax.experimental.pallas import tpu_sc as plsc

_NC = 2
_NS = 16
_NW = _NC * _NS
_B = 80
_L = 16

_GATHER_DN = lax.GatherDimensionNumbers(
    offset_dims=(), collapsed_slice_dims=(0,), start_index_map=(0,))


def _bcast_lane(vec, e):
    idx = jnp.full((_L, 1), e, jnp.int32)
    return lax.gather(vec, idx, _GATHER_DN, (1,),
                      mode=lax.GatherScatterMode.PROMISE_IN_BOUNDS)


def _sc_body(E, N, D, src_h, dst_h, val_h, emb_h, zer_h, out_h,
             src_v, dst_v, val_v, rows_v, acc_s, sem):
    c = lax.axis_index("c")
    s = lax.axis_index("s")
    wid = c * _NS + s

    rpt = N // _NS
    pltpu.sync_copy(zer_h.at[pl.ds(s * rpt, rpt)], acc_s.at[pl.ds(s * rpt, rpt)])
    plsc.subcore_barrier()

    epw = E // _NW
    nb = epw // _B
    base_w = wid * epw

    def batch(i, carry):
        base = base_w + i * _B
        pltpu.sync_copy(src_h.at[pl.ds(base, _B)], src_v)
        pltpu.sync_copy(dst_h.at[pl.ds(base, _B)], dst_v)
        pltpu.sync_copy(val_h.at[pl.ds(base, _B)], val_v)
        pltpu.async_copy(emb_h.at[src_v], rows_v, sem).wait()

        def grp(g, c2):
            vals = val_v[pl.ds(g * _L, _L)]
            for e in range(_L):
                bv = _bcast_lane(vals, e)
                r = g * _L + e
                for j in range(D // _L):
                    rows_v[r, pl.ds(j * _L, _L)] = (
                        rows_v[r, pl.ds(j * _L, _L)] * bv)
            return c2

        lax.fori_loop(0, _B // _L, grp, 0)
        pltpu.sync_copy(rows_v, acc_s.at[dst_v], add=True)
        return carry

    lax.fori_loop(0, nb, batch, 0)
    plsc.subcore_barrier()
    pltpu.sync_copy(acc_s.at[pl.ds(s * rpt, rpt)],
                    out_h.at[c, pl.ds(s * rpt, rpt)])


def _sum_body(p_ref, o_ref):
    o_ref[...] = p_ref[0] + p_ref[1]


@functools.partial(jax.jit, static_argnums=())
def _spmm(src, dst, vals, embeds, zeros):
    E = src.shape[0]
    N, D = embeds.shape
    Np = zeros.shape[0]

    sc_fn = pl.kernel(
        functools.partial(_sc_body, E, Np, D),
        out_type=jax.ShapeDtypeStruct((_NC, Np, D), jnp.float32),
        mesh=plsc.VectorSubcoreMesh(core_axis_name="c", subcore_axis_name="s"),
        scratch_types=[
            pltpu.VMEM((_B,), jnp.int32),
            pltpu.VMEM((_B,), jnp.int32),
            pltpu.VMEM((_B,), jnp.float32),
            pltpu.VMEM((_B, D), jnp.float32),
            pltpu.VMEM_SHARED((Np, D), jnp.float32),
            pltpu.SemaphoreType.DMA,
        ],
    )
    partial = sc_fn(src, dst, vals, embeds, zeros)

    R = 1024
    out = pl.pallas_call(
        _sum_body,
        grid=(Np // R,),
        in_specs=[pl.BlockSpec((_NC, R, D), lambda i: (0, i, 0))],
        out_specs=pl.BlockSpec((R, D), lambda i: (i, 0)),
        out_shape=jax.ShapeDtypeStruct((Np, D), jnp.float32),
    )(partial)
    return out[:N]


def kernel(edge_index, edge_values, embeds):
    dst = edge_index[0].astype(jnp.int32)
    src = edge_index[1].astype(jnp.int32)
    vals = edge_values.astype(jnp.float32)
    N, D = embeds.shape
    E = src.shape[0]

    chunk = _NW * _B
    Ep = ((E + chunk - 1) // chunk) * chunk
    if Ep != E:
        pad = Ep - E
        src = jnp.concatenate([src, jnp.zeros((pad,), jnp.int32)])
        dst = jnp.concatenate([dst, jnp.zeros((pad,), jnp.int32)])
        vals = jnp.concatenate([vals, jnp.zeros((pad,), jnp.float32)])

    Np = ((N + 1023) // 1024) * 1024
    zeros = jnp.zeros((Np, D), jnp.float32)
    if Np != N:
        embeds = jnp.pad(embeds, ((0, Np - N), (0, 0)))
    return _spmm(src, dst, vals, embeds, zeros)[:N]
